# baseline (speedup 1.0000x reference)
import functools

import jax
import jax.numpy as jnp
from jax import lax
from jax.experimental import pallas as pl
from jax.experimental.pallas import tpu as pltpu
from jax.experimental.pallas import tpu_sc as plsc

_VOCAB = 1000
_EMB = 16

_NC = 2
_NS = 16
_NW = _NC * _NS
_LANES = 16

_ROWS_PER_BLOCK = 512


def _argmax_body(x_ref, tok_ref):
    xb = x_ref[...]
    m = jnp.max(xb, axis=-1, keepdims=True)
    ids = lax.broadcasted_iota(jnp.int32, xb.shape, 1)
    tok_ref[0, 0, :] = jnp.min(jnp.where(xb == m, ids, _VOCAB), axis=-1)


def _tc_argmax(xf):
    rows, vocab = xf.shape
    nblk = rows // _ROWS_PER_BLOCK
    tokens = pl.pallas_call(
        _argmax_body,
        grid=(nblk,),
        in_specs=[pl.BlockSpec((_ROWS_PER_BLOCK, vocab), lambda i: (i, 0))],
        out_specs=pl.BlockSpec((1, 1, _ROWS_PER_BLOCK), lambda i: (i, 0, 0)),
        out_shape=jax.ShapeDtypeStruct((nblk, 1, _ROWS_PER_BLOCK), jnp.int32),
    )(xf)
    return tokens.reshape(rows)


def _sc_lookup(table, tokens):
    rows = tokens.shape[0]
    per_w = rows // _NW
    groups = per_w // _LANES
    mesh = plsc.VectorSubcoreMesh(core_axis_name="c", subcore_axis_name="s")

    @functools.partial(
        pl.kernel,
        mesh=mesh,
        out_type=jax.ShapeDtypeStruct((rows * _EMB,), jnp.float32),
        compiler_params=pltpu.CompilerParams(needs_layout_passes=False),
        scratch_types=[
            pltpu.VMEM((_VOCAB * _EMB,), jnp.float32),
            pltpu.VMEM((per_w,), jnp.int32),
            pltpu.VMEM((per_w * _EMB,), jnp.float32),
        ],
    )
    def k(table_hbm, tok_hbm, out_hbm, table_v, idx_v, out_v):
        wid = lax.axis_index("s") * _NC + lax.axis_index("c")
        base = wid * per_w
        pltpu.sync_copy(table_hbm, table_v)
        pltpu.sync_copy(tok_hbm.at[pl.ds(base, per_w)], idx_v)
        lane = lax.iota(jnp.int32, _LANES)

        def body(g, carry):
            tv = idx_v[pl.ds(g * _LANES, _LANES)]
            src = tv * _EMB
            dst = (g * _LANES + lane) * _EMB
            for j in range(_EMB):
                vals = plsc.load_gather(table_v, [src + j])
                plsc.store_scatter(out_v, [dst + j], vals)
            return carry

        lax.fori_loop(0, groups, body, 0)
        pltpu.sync_copy(out_v, out_hbm.at[pl.ds(base * _EMB, per_w * _EMB)])

    return k(table.reshape(_VOCAB * _EMB), tokens)


def kernel(x, table):
    b, n, vocab = x.shape
    rows = b * n
    xf = x.reshape(rows, vocab)
    tokens = _tc_argmax(xf)
    out = _sc_lookup(table, tokens)
    return out.reshape(b, n, _EMB)


# speedup vs baseline: 3.3127x; 3.3127x over previous
import functools

import jax
import jax.numpy as jnp
from jax import lax
from jax.experimental import pallas as pl
from jax.experimental.pallas import tpu as pltpu
from jax.experimental.pallas import tpu_sc as plsc

_VOCAB = 1000
_EMB = 16

_NC = 2
_NS = 16
_NW = _NC * _NS
_LANES = 16

_BBLK = 512


def _argmax_body(x_ref, tok_ref):
    xb = x_ref[0]
    m = jnp.max(xb, axis=0, keepdims=True)
    ids = lax.broadcasted_iota(jnp.int32, xb.shape, 0)
    tok_ref[0, 0, :] = jnp.min(jnp.where(xb == m, ids, _VOCAB), axis=0)


def _tc_argmax(xt):
    n, vocab, b = xt.shape
    tokens = pl.pallas_call(
        _argmax_body,
        grid=(n, b // _BBLK),
        in_specs=[pl.BlockSpec((1, vocab, _BBLK), lambda i, j: (i, 0, j))],
        out_specs=pl.BlockSpec((1, 1, _BBLK), lambda i, j: (i, 0, j)),
        out_shape=jax.ShapeDtypeStruct((n, 1, b), jnp.int32),
    )(xt)
    return tokens.reshape(n * b)


def _sc_lookup(table_flat, tokens, n, b):
    rows = n * b
    n_per_w = 2
    active = n // n_per_w
    per_w = n_per_w * b
    groups = per_w // _LANES
    mesh = plsc.VectorSubcoreMesh(core_axis_name="c", subcore_axis_name="s")

    @functools.partial(
        pl.kernel,
        mesh=mesh,
        out_type=jax.ShapeDtypeStruct((rows * _EMB,), jnp.float32),
        compiler_params=pltpu.CompilerParams(needs_layout_passes=False),
        scratch_types=[
            pltpu.VMEM((_VOCAB * _EMB,), jnp.float32),
            pltpu.VMEM((per_w,), jnp.int32),
            pltpu.VMEM((per_w * _EMB,), jnp.float32),
        ],
    )
    def k(table_hbm, tok_hbm, out_hbm, table_v, idx_v, out_v):
        wid = lax.axis_index("s") * _NC + lax.axis_index("c")

        @pl.when(wid < active)
        def _():
            pltpu.sync_copy(table_hbm, table_v)
            pltpu.sync_copy(tok_hbm.at[pl.ds(wid * per_w, per_w)], idx_v)

            def body(g, carry):
                tv = idx_v[pl.ds(g * _LANES, _LANES)]
                src = tv * _EMB
                gpn = b // _LANES
                npr = g // gpn
                b0 = (g - npr * gpn) * _LANES
                dbase = npr * (_EMB * b) + b0
                for j in range(_EMB):
                    vals = plsc.load_gather(table_v, [src + j])
                    out_v[pl.ds(dbase + j * b, _LANES)] = vals
                return carry

            lax.fori_loop(0, groups, body, 0)
            pltpu.sync_copy(
                out_v, out_hbm.at[pl.ds(wid * per_w * _EMB, per_w * _EMB)]
            )

    return k(table_flat, tokens)


def kernel(x, table):
    b, n, vocab = x.shape
    xt = jnp.transpose(x, (1, 2, 0))
    tokens = _tc_argmax(xt)
    out_flat = _sc_lookup(table.reshape(vocab * _EMB), tokens, n, b)
    out_t = out_flat.reshape(n, _EMB, b)
    return jnp.transpose(out_t, (2, 0, 1))


# speedup vs baseline: 4.7685x; 1.4395x over previous
import functools

import jax
import jax.numpy as jnp
from jax import lax
from jax.experimental import pallas as pl
from jax.experimental.pallas import tpu as pltpu
from jax.experimental.pallas import tpu_sc as plsc

_VOCAB = 1000
_EMB = 16

_NC = 2
_NS = 16
_NW = _NC * _NS
_LANES = 16

_BBLK = 1024
_NBLK = 5

_CHUNKS = 2


def _argmax_body(x_ref, tok_ref):
    xb = x_ref[...]
    m = jnp.max(xb, axis=1, keepdims=True)
    ids = lax.broadcasted_iota(jnp.int32, xb.shape, 1)
    tok_ref[...] = jnp.min(jnp.where(xb == m, ids, _VOCAB), axis=1, keepdims=True)


def _tc_argmax_chunk(xt, c, npc):
    n, vocab, b = xt.shape
    n0 = c * npc
    tokens = pl.pallas_call(
        _argmax_body,
        grid=(npc // _NBLK, b // _BBLK),
        in_specs=[
            pl.BlockSpec(
                (_NBLK, vocab, _BBLK),
                lambda i, j, n0=n0: (n0 // _NBLK + i, 0, j),
            )
        ],
        out_specs=pl.BlockSpec((_NBLK, 1, _BBLK), lambda i, j: (i, 0, j)),
        out_shape=jax.ShapeDtypeStruct((npc, 1, b), jnp.int32),
    )(xt)
    return tokens.reshape(npc * b)


def _sc_lookup(table_flat, tokens, npc, b):
    per_w = b
    groups = per_w // _LANES
    mesh = plsc.VectorSubcoreMesh(core_axis_name="c", subcore_axis_name="s")

    @functools.partial(
        pl.kernel,
        mesh=mesh,
        out_type=jax.ShapeDtypeStruct((npc * _EMB * b,), jnp.float32),
        compiler_params=pltpu.CompilerParams(needs_layout_passes=False),
        scratch_types=[
            pltpu.VMEM((_VOCAB * _EMB,), jnp.float32),
            pltpu.VMEM((per_w,), jnp.int32),
            pltpu.VMEM((per_w * _EMB,), jnp.float32),
        ],
    )
    def k(table_hbm, tok_hbm, out_hbm, table_v, idx_v, out_v):
        wid = lax.axis_index("s") * _NC + lax.axis_index("c")

        @pl.when(wid < npc)
        def _():
            pltpu.sync_copy(table_hbm, table_v)
            pltpu.sync_copy(tok_hbm.at[pl.ds(wid * per_w, per_w)], idx_v)

            def body(g, carry):
                tv = idx_v[pl.ds(g * _LANES, _LANES)]
                src = tv * _EMB
                b0 = g * _LANES
                for j in range(_EMB):
                    vals = plsc.load_gather(table_v, [src + j])
                    out_v[pl.ds(j * b + b0, _LANES)] = vals
                return carry

            lax.fori_loop(0, groups, body, 0)
            pltpu.sync_copy(
                out_v, out_hbm.at[pl.ds(wid * per_w * _EMB, per_w * _EMB)]
            )

    return k(table_flat, tokens)


def kernel(x, table):
    b, n, vocab = x.shape
    xt = jnp.transpose(x, (1, 2, 0))
    table_flat = table.reshape(vocab * _EMB)
    npc = n // _CHUNKS
    parts = []
    for c in range(_CHUNKS):
        tok_c = _tc_argmax_chunk(xt, c, npc)
        out_c = _sc_lookup(table_flat, tok_c, npc, b)
        parts.append(out_c.reshape(npc, _EMB, b))
    out_t = jnp.concatenate(parts, axis=0) if len(parts) > 1 else parts[0]
    return jnp.transpose(out_t, (2, 0, 1))
